# baseline (speedup 1.0000x reference)
import functools

import jax
import jax.numpy as jnp
from jax import lax
from jax.experimental import pallas as pl
from jax.experimental.pallas import tpu as pltpu
from jax.experimental.pallas import tpu_sc as plsc

_E = 64
_K = 2
_H = 1024
_F = 512
_T = 256
_LANES = 16
_NW = 16
_EPG = 4


def _logits_body(x_ref, w_ref, lt_ref):
    x = x_ref[...]
    w = w_ref[...]
    logits = jax.lax.dot_general(
        x, w, (((1,), (1,)), ((), ())),
        preferred_element_type=jnp.float32)
    lt_ref[...] = logits.T


def _sc_route_body(lt_hbm, pk_hbm, w1_hbm, stage, o_pk, o_w1):
    wid = lax.axis_index("c") * 16 + lax.axis_index("s")

    @pl.when(wid < _NW)
    def _():
        base = wid * _LANES
        pltpu.sync_copy(lt_hbm, stage)
        neg = jnp.full((_LANES,), -3e38, jnp.float32)
        zero_i = jnp.zeros((_LANES,), jnp.int32)
        m1, m2, i1, i2 = neg, neg, zero_i, zero_i
        for e in range(_E):
            ev = jnp.full((_LANES,), e, jnp.int32)
            v = stage[e, pl.ds(base, _LANES)]
            gt1 = v > m1
            gt2 = v > m2
            m2 = jnp.where(gt1, m1, jnp.where(gt2, v, m2))
            i2 = jnp.where(gt1, i1, jnp.where(gt2, ev, i2))
            m1 = jnp.where(gt1, v, m1)
            i1 = jnp.where(gt1, ev, i1)
        o_pk[...] = i1 * _E + i2
        o_w1[...] = 1.0 / (1.0 + jnp.exp(m2 - m1))
        pltpu.sync_copy(o_pk, pk_hbm.at[pl.ds(base, _LANES)])
        pltpu.sync_copy(o_w1, w1_hbm.at[pl.ds(base, _LANES)])


def _moe_body(pk_ref, w1_ref, x_ref, up_ref, dn_ref, o_ref):
    g = pl.program_id(0)

    @pl.when(g == 0)
    def _init():
        o_ref[...] = jnp.zeros_like(o_ref)

    xb = x_ref[...].astype(jnp.bfloat16)
    pk = pk_ref[...]
    w1 = w1_ref[...]
    i1 = pk // _E
    i2 = pk - i1 * _E
    acc = jnp.zeros_like(o_ref)
    for j in range(_EPG):
        e = g * _EPG + j
        up = up_ref[j].astype(jnp.bfloat16)
        h = jnp.dot(xb, up, preferred_element_type=jnp.float32)
        h1 = h[:, :_F]
        h2 = h[:, _F:]
        act = (h1 * jax.nn.sigmoid(h1) * h2).astype(jnp.bfloat16)
        dn = dn_ref[j].astype(jnp.bfloat16)
        y = jnp.dot(act, dn, preferred_element_type=jnp.float32)
        col = (jnp.where(i1 == e, w1, 0.0)
               + jnp.where(i2 == e, 1.0 - w1, 0.0))
        acc += col * y
    o_ref[...] += acc


def _sc_route(logits):
    mesh = plsc.VectorSubcoreMesh(core_axis_name="c", subcore_axis_name="s")
    f = pl.kernel(
        _sc_route_body,
        mesh=mesh,
        out_type=(
            jax.ShapeDtypeStruct((_T,), jnp.int32),
            jax.ShapeDtypeStruct((_T,), jnp.float32),
        ),
        scratch_types=[
            pltpu.VMEM((_E, _T), jnp.float32),
            pltpu.VMEM((_LANES,), jnp.int32),
            pltpu.VMEM((_LANES,), jnp.float32),
        ],
    )
    return f(logits)


def kernel(x, router_w, up_proj, down_proj):
    b, s, h = x.shape
    xf = x.reshape(-1, h)
    t = xf.shape[0]

    logits_t = pl.pallas_call(
        _logits_body,
        out_shape=jax.ShapeDtypeStruct((_E, t), jnp.float32),
    )(xf, router_w)

    pk, w1 = _sc_route(logits_t)
    pk = pk.reshape(t, 1)
    w1 = w1.reshape(t, 1)

    out = pl.pallas_call(
        _moe_body,
        grid=(_E // _EPG,),
        in_specs=[
            pl.BlockSpec((t, 1), lambda e: (0, 0)),
            pl.BlockSpec((t, 1), lambda e: (0, 0)),
            pl.BlockSpec((t, h), lambda e: (0, 0)),
            pl.BlockSpec((_EPG, _H, 2 * _F), lambda e: (e, 0, 0)),
            pl.BlockSpec((_EPG, _F, _H), lambda e: (e, 0, 0)),
        ],
        out_specs=pl.BlockSpec((t, h), lambda e: (0, 0)),
        out_shape=jax.ShapeDtypeStruct((t, h), jnp.float32),
    )(pk, w1, xf, up_proj, down_proj)

    return out.reshape(b, s, h)


# speedup vs baseline: 1.0220x; 1.0220x over previous
import functools

import jax
import jax.numpy as jnp
from jax import lax
from jax.experimental import pallas as pl
from jax.experimental.pallas import tpu as pltpu
from jax.experimental.pallas import tpu_sc as plsc

_E = 64
_K = 2
_H = 1024
_F = 512
_T = 256
_LANES = 16
_NW = 16
_EPG = 2


def _logits_body(x_ref, w_ref, lt_ref):
    x = x_ref[...]
    w = w_ref[...]
    logits = jax.lax.dot_general(
        x, w, (((1,), (1,)), ((), ())),
        preferred_element_type=jnp.float32)
    lt_ref[...] = logits.T


def _sc_route_body(lt_hbm, pk_hbm, w1_hbm, stage, o_pk, o_w1):
    wid = lax.axis_index("c") * 16 + lax.axis_index("s")

    @pl.when(wid < _NW)
    def _():
        base = wid * _LANES
        pltpu.sync_copy(lt_hbm, stage)
        neg = jnp.full((_LANES,), -3e38, jnp.float32)
        zero_i = jnp.zeros((_LANES,), jnp.int32)
        m1, m2, i1, i2 = neg, neg, zero_i, zero_i
        for e in range(_E):
            ev = jnp.full((_LANES,), e, jnp.int32)
            v = stage[e, pl.ds(base, _LANES)]
            gt1 = v > m1
            gt2 = v > m2
            m2 = jnp.where(gt1, m1, jnp.where(gt2, v, m2))
            i2 = jnp.where(gt1, i1, jnp.where(gt2, ev, i2))
            m1 = jnp.where(gt1, v, m1)
            i1 = jnp.where(gt1, ev, i1)
        o_pk[...] = i1 * _E + i2
        o_w1[...] = 1.0 / (1.0 + jnp.exp(m2 - m1))
        pltpu.sync_copy(o_pk, pk_hbm.at[pl.ds(base, _LANES)])
        pltpu.sync_copy(o_w1, w1_hbm.at[pl.ds(base, _LANES)])


def _moe_body(pk_ref, w1_ref, x_ref, up_ref, dn_ref, o_ref):
    g = pl.program_id(0)

    @pl.when(g == 0)
    def _init():
        o_ref[...] = jnp.zeros_like(o_ref)

    xb = x_ref[...].astype(jnp.bfloat16)
    pk = pk_ref[...]
    w1 = w1_ref[...]
    i1 = pk // _E
    i2 = pk - i1 * _E
    acc = jnp.zeros_like(o_ref)
    for j in range(_EPG):
        e = g * _EPG + j
        up = up_ref[j].astype(jnp.bfloat16)
        h = jnp.dot(xb, up, preferred_element_type=jnp.float32)
        h1 = h[:, :_F]
        h2 = h[:, _F:]
        act = (h1 * jax.nn.sigmoid(h1) * h2).astype(jnp.bfloat16)
        dn = dn_ref[j].astype(jnp.bfloat16)
        y = jnp.dot(act, dn, preferred_element_type=jnp.float32)
        col = (jnp.where(i1 == e, w1, 0.0)
               + jnp.where(i2 == e, 1.0 - w1, 0.0))
        acc += col * y
    o_ref[...] += acc


def _sc_route(logits):
    mesh = plsc.VectorSubcoreMesh(core_axis_name="c", subcore_axis_name="s")
    f = pl.kernel(
        _sc_route_body,
        mesh=mesh,
        out_type=(
            jax.ShapeDtypeStruct((_T,), jnp.int32),
            jax.ShapeDtypeStruct((_T,), jnp.float32),
        ),
        scratch_types=[
            pltpu.VMEM((_E, _T), jnp.float32),
            pltpu.VMEM((_LANES,), jnp.int32),
            pltpu.VMEM((_LANES,), jnp.float32),
        ],
    )
    return f(logits)


def kernel(x, router_w, up_proj, down_proj):
    b, s, h = x.shape
    xf = x.reshape(-1, h)
    t = xf.shape[0]

    logits_t = pl.pallas_call(
        _logits_body,
        out_shape=jax.ShapeDtypeStruct((_E, t), jnp.float32),
    )(xf, router_w)

    pk, w1 = _sc_route(logits_t)
    pk = pk.reshape(t, 1)
    w1 = w1.reshape(t, 1)

    out = pl.pallas_call(
        _moe_body,
        grid=(_E // _EPG,),
        in_specs=[
            pl.BlockSpec((t, 1), lambda e: (0, 0)),
            pl.BlockSpec((t, 1), lambda e: (0, 0)),
            pl.BlockSpec((t, h), lambda e: (0, 0)),
            pl.BlockSpec((_EPG, _H, 2 * _F), lambda e: (e, 0, 0)),
            pl.BlockSpec((_EPG, _F, _H), lambda e: (e, 0, 0)),
        ],
        out_specs=pl.BlockSpec((t, h), lambda e: (0, 0)),
        out_shape=jax.ShapeDtypeStruct((t, h), jnp.float32),
    )(pk, w1, xf, up_proj, down_proj)

    return out.reshape(b, s, h)
